# Initial kernel scaffold; baseline (speedup 1.0000x reference)
import jax
import jax.numpy as jnp
from jax.experimental import pallas as pl


def kernel(x, hyperedge_index, hyperedge_attr):
    raise NotImplementedError("write your pallas kernel here")


# speedup vs baseline: 20.8523x; 20.8523x over previous
import functools

import jax
import jax.numpy as jnp
from jax.experimental import pallas as pl
from jax.experimental.pallas import tpu as pltpu

N_NODES = 10000
N_EDGES = 2000
NP = 10112
EP = 2048
NB = 1264
EB = 256
STEPS = 16
SCALES = [0, 1, 2, 4, 8, 16]


def _hilo(v):
    hi = v.astype(jnp.bfloat16)
    lo = (v - hi.astype(jnp.float32)).astype(jnp.bfloat16)
    return jnp.concatenate([hi, lo], axis=1)


def _degrees_body(b_ref, bt_ref, dvinv_ref, dheinv_ref):
    b = pl.program_id(0)
    ones_e = jnp.ones((EP, 128), jnp.bfloat16)
    ones_n = jnp.ones((NP, 128), jnp.bfloat16)
    dv = jax.lax.dot_general(bt_ref[...], ones_e, (((1,), (0,)), ((), ())),
                             preferred_element_type=jnp.float32)
    dvinv_ref[pl.ds(b * NB, NB), :] = jnp.where(dv > 0, 1.0 / dv, 0.0)
    dhe = jax.lax.dot_general(b_ref[...], ones_n, (((1,), (0,)), ((), ())),
                              preferred_element_type=jnp.float32)
    dheinv_ref[pl.ds(b * EB, EB), :] = jnp.where(dhe > 0, 1.0 / dhe, 0.0)


def _degrees(B, BT):
    return pl.pallas_call(
        _degrees_body,
        grid=(8,),
        in_specs=[
            pl.BlockSpec((EB, NP), lambda b: (b, 0)),
            pl.BlockSpec((NB, EP), lambda b: (b, 0)),
        ],
        out_specs=[
            pl.BlockSpec((NP, 128), lambda b: (0, 0)),
            pl.BlockSpec((EP, 128), lambda b: (0, 0)),
        ],
        out_shape=[
            jax.ShapeDtypeStruct((NP, 128), jnp.float32),
            jax.ShapeDtypeStruct((EP, 128), jnp.float32),
        ],
    )(B, BT)


def _diffusion_body(b_ref, bt_ref, x_ref, dvinv_ref, dheinv_ref,
                    nlev_ref, elev_ref, xcur, yhl, e_raw, e2hl):
    k = pl.program_id(0)
    d = pl.program_id(1)
    b = pl.program_id(2)

    @pl.when((k == 0) & (d == 0) & (b == 0))
    def _init():
        xcur[...] = x_ref[...]

    @pl.when(d == 0)
    def _edge_dir():
        @pl.when(b == 0)
        def _scale():
            yhl[...] = _hilo(xcur[...] * dvinv_ref[...])

        part = jax.lax.dot_general(b_ref[...], yhl[...],
                                   (((1,), (0,)), ((), ())),
                                   preferred_element_type=jnp.float32)
        e_rows = part[:, :128] + part[:, 128:]
        e_raw[pl.ds(b * EB, EB), :] = e_rows

        @pl.when(b == 7)
        def _finish_edges():
            er = e_raw[...]
            elev_ref[...] = er[None]
            e2hl[...] = _hilo(er * dheinv_ref[...])

    @pl.when(d == 1)
    def _node_dir():
        part = jax.lax.dot_general(bt_ref[...], e2hl[...],
                                   (((1,), (0,)), ((), ())),
                                   preferred_element_type=jnp.float32)
        x_rows = part[:, :128] + part[:, 128:]
        nlev_ref[...] = x_rows[None]
        xcur[pl.ds(b * NB, NB), :] = x_rows


def _diffusion(B, BT, x_pad, dvinv, dheinv):
    return pl.pallas_call(
        _diffusion_body,
        grid=(STEPS, 2, 8),
        in_specs=[
            pl.BlockSpec((EB, NP), lambda k, d, b: (jnp.where(d == 0, b, 7), 0)),
            pl.BlockSpec((NB, EP), lambda k, d, b: (jnp.where(d == 1, b, 0), 0)),
            pl.BlockSpec((NP, 128), lambda k, d, b: (0, 0)),
            pl.BlockSpec((NP, 128), lambda k, d, b: (0, 0)),
            pl.BlockSpec((EP, 128), lambda k, d, b: (0, 0)),
        ],
        out_specs=[
            pl.BlockSpec((1, NB, 128), lambda k, d, b: (k, b, 0)),
            pl.BlockSpec((1, EP, 128), lambda k, d, b: (k, 0, 0)),
        ],
        out_shape=[
            jax.ShapeDtypeStruct((STEPS, NP, 128), jnp.float32),
            jax.ShapeDtypeStruct((STEPS, EP, 128), jnp.float32),
        ],
        scratch_shapes=[
            pltpu.VMEM((NP, 128), jnp.float32),
            pltpu.VMEM((NP, 256), jnp.bfloat16),
            pltpu.VMEM((EP, 128), jnp.float32),
            pltpu.VMEM((EP, 256), jnp.bfloat16),
        ],
    )(B, BT, x_pad, dvinv, dheinv)


def _interleave_mat():
    row = jax.lax.broadcasted_iota(jnp.int32, (128, 256), 0)
    col = jax.lax.broadcasted_iota(jnp.int32, (128, 256), 1)
    return (jnp.where(col == 2 * row, 1.0, 0.0)
            - jnp.where(col == 2 * row + 1, 1.0, 0.0)).astype(jnp.float32)


def _wavelet_body(l0_ref, l1_ref, l2_ref, l4_ref, l8_ref, l16_ref, out_ref):
    R = _interleave_mat()
    levels = [l0_ref[...], l1_ref[0], l2_ref[0], l4_ref[0], l8_ref[0],
              l16_ref[0]]
    coeffs = [levels[i] - levels[i + 1] for i in range(5)] + [levels[5]]
    for w, c in enumerate(coeffs):
        v = jax.lax.dot_general(c, R, (((1,), (0,)), ((), ())),
                                preferred_element_type=jnp.float32)
        out_ref[:, 256 * w:256 * (w + 1)] = jnp.maximum(v, 0.0)


def _wavelet(level0, levels, rows, row_block):
    nb = rows // row_block
    lvl_specs = [
        pl.BlockSpec((1, row_block, 128),
                     functools.partial(lambda s, i: (s, i, 0), k - 1))
        for k in SCALES[1:]
    ]
    return pl.pallas_call(
        _wavelet_body,
        grid=(nb,),
        in_specs=[pl.BlockSpec((row_block, 128), lambda i: (i, 0))] + lvl_specs,
        out_specs=pl.BlockSpec((row_block, 1536), lambda i: (i, 0)),
        out_shape=jax.ShapeDtypeStruct((rows, 1536), jnp.float32),
    )(level0, *([levels] * 5))


def kernel(x, hyperedge_index, hyperedge_attr):
    src = hyperedge_index[0].astype(jnp.int32)
    dst = hyperedge_index[1].astype(jnp.int32)
    x_pad = jnp.pad(x, ((0, NP - N_NODES), (0, 0)))
    attr_pad = jnp.pad(hyperedge_attr, ((0, EP - N_EDGES), (0, 0)))

    B = jnp.zeros((EP, NP), jnp.float32).at[dst, src].add(1.0).astype(jnp.bfloat16)
    BT = jnp.zeros((NP, EP), jnp.float32).at[src, dst].add(1.0).astype(jnp.bfloat16)

    dvinv, dheinv = _degrees(B, BT)
    node_levels, edge_levels = _diffusion(B, BT, x_pad, dvinv, dheinv)
    s_nodes = _wavelet(x_pad, node_levels, NP, NB)
    s_edges = _wavelet(attr_pad, edge_levels, EP, EP)
    return s_nodes[:N_NODES], s_edges[:N_EDGES]
